# baseline (speedup 1.0000x reference)
import functools

import jax
import jax.numpy as jnp
from jax import lax
from jax.experimental import pallas as pl
from jax.experimental.pallas import tpu as pltpu
from jax.experimental.pallas import tpu_sc as plsc

V = 100000
D = 64
B = 1024
C = 50

L = 16
NC = 2
NS = 16
NW = NC * NS
B_PER_W = B // NW
R_PER_W = B_PER_W * C
NCHUNK = 16
CH = R_PER_W // NCHUNK
NG = R_PER_W // L

NV = 2048
NBLK = (V + NV - 1) // NV


def _pool_sc(idx_r, emb_table):
    mesh = plsc.VectorSubcoreMesh(core_axis_name="c", subcore_axis_name="s")

    @functools.partial(
        pl.kernel,
        mesh=mesh,
        out_type=jax.ShapeDtypeStruct((B, D), jnp.float32),
        scratch_types=[
            pltpu.VMEM((NCHUNK, CH), jnp.int32),
            pltpu.VMEM((R_PER_W, D), jnp.float32),
            pltpu.VMEM((B_PER_W, D), jnp.float32),
            pltpu.SemaphoreType.DMA,
        ],
        compiler_params=pltpu.CompilerParams(
            needs_layout_passes=False, use_tc_tiling_on_sc=False
        ),
    )
    def k(idx_hbm, table_hbm, out_hbm, idx_v, rows_v, pooled_v, sem):
        wid = lax.axis_index("s") * NC + lax.axis_index("c")
        pltpu.sync_copy(idx_hbm.at[wid], idx_v)
        cps = [
            pltpu.async_copy(
                table_hbm.at[idx_v.at[j]], rows_v.at[pl.ds(j * CH, CH)], sem
            )
            for j in range(NCHUNK)
        ]
        for cp in cps:
            cp.wait()

        def b_body(lb, carry):
            rbase = lb * C

            def row_update(r, accs):
                vs = [rows_v[r, pl.ds(u * L, L)] for u in range(D // L)]
                w = vs[0] * vs[0]
                for v in vs[1:]:
                    w = w + v * v
                s = jnp.sum(w)
                i32 = lax.bitcast_convert_type(s, jnp.int32)
                y = lax.bitcast_convert_type(
                    jnp.int32(0x5F3759DF) - (i32 >> 1), jnp.float32
                )
                for _ in range(3):
                    y = y * (1.5 - 0.5 * s * y * y)
                scv = jnp.full((L,), jnp.where(s > 1.0, y, jnp.float32(1.0)))
                return tuple(a + scv * v for a, v in zip(accs, vs))

            def c_body(cc, accs):
                accs = row_update(rbase + cc * 2, accs)
                return row_update(rbase + cc * 2 + 1, accs)

            zero = jnp.zeros((L,), jnp.float32)
            accs = lax.fori_loop(0, C // 2, c_body, (zero, zero, zero, zero))
            inv = jnp.float32(1.0 / C)
            for u in range(4):
                pooled_v[lb, pl.ds(u * L, L)] = accs[u] * inv
            return carry

        lax.fori_loop(0, B_PER_W, b_body, 0)
        pltpu.sync_copy(pooled_v, out_hbm.at[pl.ds(wid * B_PER_W, B_PER_W)])

    return k(idx_r, emb_table)


def _project_tc(pooled, lin_w, lin_b2):
    def mm(x_ref, w_ref, b_ref, o_ref):
        acc = lax.dot_general(
            x_ref[...], w_ref[...],
            (((1,), (1,)), ((), ())),
            preferred_element_type=jnp.float32,
        )
        o_ref[...] = acc + b_ref[...]

    return pl.pallas_call(
        mm,
        grid=(NBLK,),
        in_specs=[
            pl.BlockSpec((B, D), lambda i: (0, 0)),
            pl.BlockSpec((NV, D), lambda i: (i, 0)),
            pl.BlockSpec((1, NV), lambda i: (0, i)),
        ],
        out_specs=pl.BlockSpec((B, NV), lambda i: (0, i)),
        out_shape=jax.ShapeDtypeStruct((B, V), jnp.float32),
        compiler_params=pltpu.CompilerParams(
            dimension_semantics=("arbitrary",)
        ),
    )(pooled, lin_w, lin_b2)


def kernel(inputs_, emb_table, lin_w, lin_b):
    idx = inputs_.astype(jnp.int32).reshape(NW, NCHUNK, CH)
    pooled = _pool_sc(idx, emb_table)
    return _project_tc(pooled, lin_w, lin_b.reshape(1, V))
